# baseline (speedup 1.0000x reference)
import functools

import jax
import jax.numpy as jnp
import numpy as np
from jax.experimental import pallas as pl

_STRIDES = (8, 16, 32, 64, 128)
_HWS = (64, 32, 16, 8, 4)
_A = 9
_C = 80
_IMG = 512.0
_DWH_CLAMP = 4.135
_IOU_THR = 0.5
_POST_NMS = 100
_NCAND = 384
_NMS_N = 384

_KS = tuple(hw * hw * _A for hw in _HWS)
_OFFS = tuple(int(x) for x in np.concatenate([[0], np.cumsum(_KS)]))
_KTOT = _OFFS[-1]


def _make_anchor_table():
    tabs = []
    for hw, stride in zip(_HWS, _STRIDES):
        ratios = np.array([0.5, 1.0, 2.0])
        scales = np.array([2.0 ** 0.0, 2.0 ** (1.0 / 3.0), 2.0 ** (2.0 / 3.0)])
        base = 4.0 * stride
        ws, hs = [], []
        for r in ratios:
            for s in scales:
                size = base * s
                ws.append(size * np.sqrt(1.0 / r))
                hs.append(size * np.sqrt(r))
        ws = np.array(ws)
        hs = np.array(hs)
        xs = (np.arange(hw) + 0.5) * stride
        ys = (np.arange(hw) + 0.5) * stride
        cx, cy = np.meshgrid(xs, ys)
        cx = cx.reshape(-1, 1)
        cy = cy.reshape(-1, 1)
        x1 = cx - ws / 2.0
        y1 = cy - hs / 2.0
        x2 = cx + ws / 2.0
        y2 = cy + hs / 2.0
        tabs.append(np.stack([x1, y1, x2, y2], axis=-1).reshape(-1, 4))
    return np.concatenate(tabs, axis=0).astype(np.float32)


_ANCHOR_TABLE = _make_anchor_table()


def _amax_body(x_ref, o_ref):
    x = x_ref[0]
    outs = []
    for j in range(_A):
        outs.append(jnp.max(x[j * _C:(j + 1) * _C, :], axis=0))
    o_ref[0] = jnp.stack(outs, axis=0)


def _anchor_max(cls_flat, p):
    bp = min(p, 512)
    grid = (cls_flat.shape[0], p // bp)
    return pl.pallas_call(
        _amax_body,
        grid=grid,
        in_specs=[pl.BlockSpec((1, _A * _C, bp), lambda b, q: (b, 0, q))],
        out_specs=pl.BlockSpec((1, _A, bp), lambda b, q: (b, 0, q)),
        out_shape=jax.ShapeDtypeStruct((cls_flat.shape[0], _A, p), jnp.float32),
    )(cls_flat)


def _radix_select_idx(v, valid, k):
    rr = v.shape[0]
    i32 = jax.lax.bitcast_convert_type(v, jnp.int32)
    key = i32 ^ jnp.bitwise_and(jnp.right_shift(i32, 31),
                                jnp.int32(0x7FFFFFFF))
    ukey = jax.lax.bitcast_convert_type(key, jnp.uint32) ^ jnp.uint32(
        0x80000000)
    row_i = jax.lax.broadcasted_iota(jnp.int32, (rr, 128), 0)
    col_i = jax.lax.broadcasted_iota(jnp.int32, (rr, 128), 1)
    flat = row_i * 128 + col_i
    ukey = jnp.where(flat < valid, ukey, jnp.uint32(0))

    t = jnp.uint32(0)
    for bit in range(31, -1, -1):
        cand = t | jnp.uint32(1 << bit)
        cnt = jnp.sum((ukey >= cand).astype(jnp.float32))
        t = jnp.where(cnt >= k, cand, t)

    gt = ukey > t
    eq = ukey == t
    quota = jnp.float32(k) - jnp.sum(gt.astype(jnp.float32))

    up_tri = (jax.lax.broadcasted_iota(jnp.int32, (128, 128), 0)
              < jax.lax.broadcasted_iota(jnp.int32, (128, 128), 1)
              ).astype(jnp.float32)
    low_r = (jax.lax.broadcasted_iota(jnp.int32, (rr, rr), 0)
             < jax.lax.broadcasted_iota(jnp.int32, (rr, rr), 1)
             ).astype(jnp.float32)

    eq_f = eq.astype(jnp.float32)
    e_in = jax.lax.dot(eq_f, up_tri, preferred_element_type=jnp.float32)
    s_eq = jnp.sum(eq_f, axis=1, keepdims=True)
    s_pre = jnp.transpose(jnp.sum(low_r * s_eq, axis=0, keepdims=True))
    eqrank = s_pre + e_in
    sel = gt | (eq & (eqrank < quota))

    sel_f = sel.astype(jnp.float32)
    p_in = jax.lax.dot(sel_f, up_tri, preferred_element_type=jnp.float32)
    t_cnt = jnp.sum(sel_f, axis=1, keepdims=True)
    t_pre = jnp.transpose(jnp.sum(low_r * t_cnt, axis=0, keepdims=True))

    trow = jnp.transpose(t_pre)
    pcol = jax.lax.broadcasted_iota(
        jnp.int32, (k, 1), 0).astype(jnp.float32)
    row_of_p = jnp.sum((trow <= pcol).astype(jnp.float32),
                       axis=1, keepdims=True) - 1.0
    ridx = jax.lax.broadcasted_iota(
        jnp.int32, (k, rr), 1).astype(jnp.float32)
    oh = (ridx == row_of_p).astype(jnp.float32)
    t_g = jnp.sum(oh * trow, axis=1, keepdims=True)
    o_p = pcol - t_g
    w_g = jax.lax.dot(oh, p_in, preferred_element_type=jnp.float32)
    sel_g = jax.lax.dot(oh, sel_f, preferred_element_type=jnp.float32)
    lane = jax.lax.broadcasted_iota(
        jnp.int32, (k, 128), 1).astype(jnp.float32)
    hit = sel_g * (w_g == o_p).astype(jnp.float32)
    out = jnp.sum(hit * (row_of_p * 128.0 + lane), axis=1, keepdims=True)
    return out.astype(jnp.int32)


def _select_body(v_ref, o_ref):
    idx = _radix_select_idx(v_ref[0], _KTOT, _NCAND)
    o_ref[0] = jnp.transpose(idx)


def _select_anchors(maxes):
    b = maxes.shape[0]
    npad = 128 * ((_KTOT + 127) // 128) - _KTOT
    mp = jnp.concatenate(
        [maxes, jnp.zeros((b, npad), jnp.float32)], axis=1)
    mp = mp.reshape(b, (_KTOT + npad) // 128, 128)
    out = pl.pallas_call(
        _select_body,
        grid=(b,),
        in_specs=[pl.BlockSpec(
            (1, mp.shape[1], 128), lambda i: (i, 0, 0))],
        out_specs=pl.BlockSpec((1, 1, _NCAND), lambda i: (i, 0, 0)),
        out_shape=jax.ShapeDtypeStruct((b, 1, _NCAND), jnp.int32),
    )(mp)
    return out.reshape(b, _NCAND)


def _nms_body(dx_ref, dy_ref, dw_ref, dh_ref, ax1_ref, ay1_ref, ax2_ref,
              ay2_ref, s_ref, lab_ref, o_ref):
    dx = dx_ref[0]
    dy = dy_ref[0]
    dw = jnp.clip(dw_ref[0], -_DWH_CLAMP, _DWH_CLAMP)
    dh = jnp.clip(dh_ref[0], -_DWH_CLAMP, _DWH_CLAMP)
    ax1 = ax1_ref[0]
    ay1 = ay1_ref[0]
    ax2 = ax2_ref[0]
    ay2 = ay2_ref[0]
    s = s_ref[0]
    lab = lab_ref[0]

    wa = ax2 - ax1
    ha = ay2 - ay1
    xa = ax1 + 0.5 * wa
    ya = ay1 + 0.5 * ha
    px = dx * wa + xa
    py = dy * ha + ya
    pw = jnp.exp(dw) * wa
    ph = jnp.exp(dh) * ha
    x1 = jnp.clip(px - 0.5 * pw, 0.0, _IMG)
    y1 = jnp.clip(py - 0.5 * ph, 0.0, _IMG)
    x2 = jnp.clip(px + 0.5 * pw, 0.0, _IMG)
    y2 = jnp.clip(py + 0.5 * ph, 0.0, _IMG)

    off = lab * (2.0 * _IMG)
    ox1 = x1 + off
    oy1 = y1 + off
    ox2 = x2 + off
    oy2 = y2 + off

    area = (x2 - x1) * (y2 - y1)

    cx1 = jnp.transpose(ox1)
    cy1 = jnp.transpose(oy1)
    cx2 = jnp.transpose(ox2)
    cy2 = jnp.transpose(oy2)
    carea = jnp.transpose(area)
    cs = jnp.transpose(s)

    ltx = jnp.maximum(cx1, ox1)
    lty = jnp.maximum(cy1, oy1)
    rbx = jnp.minimum(cx2, ox2)
    rby = jnp.minimum(cy2, oy2)
    w = jnp.clip(rbx - ltx, 0.0, None)
    h = jnp.clip(rby - lty, 0.0, None)
    inter = w * h
    union = carea + area - inter
    iou = inter / jnp.clip(union, 1e-6, None)

    higher = cs > s
    sup = jnp.any(higher & (iou > _IOU_THR), axis=0, keepdims=True)

    s2 = jnp.where(sup, -1e9, s)

    in_a = jnp.logical_and(jnp.logical_not(sup), s > 0.0)
    n = s.shape[1]
    ia_row = in_a
    ia_col = jnp.transpose(in_a)
    row_j = jax.lax.broadcasted_iota(jnp.int32, (n, n), 1)
    col_i = jax.lax.broadcasted_iota(jnp.int32, (n, n), 0)
    before = jnp.logical_or(
        jnp.logical_and(ia_col, jnp.logical_not(ia_row)),
        jnp.logical_and(ia_col == ia_row, col_i < row_j),
    )
    pos = jnp.sum(before.astype(jnp.float32), axis=0, keepdims=True)

    sel = (pos == col_i.astype(jnp.float32)).astype(jnp.float32)

    def compact(row):
        return jnp.sum(sel * row, axis=1, keepdims=True)

    out = jnp.concatenate(
        [compact(x1), compact(y1), compact(x2), compact(y2),
         compact(s2), compact(lab), jnp.zeros((n, 2), jnp.float32)], axis=1)
    o_ref[0] = out


def _nms_topk(dx, dy, dw, dh, ax1, ay1, ax2, ay2, s, lab):
    b, n = s.shape
    r3 = lambda a: a.reshape(b, 1, n)
    args = [r3(a) for a in (dx, dy, dw, dh, ax1, ay1, ax2, ay2, s, lab)]
    spec = pl.BlockSpec((1, 1, n), lambda i: (i, 0, 0))
    return pl.pallas_call(
        _nms_body,
        grid=(b,),
        in_specs=[spec] * 10,
        out_specs=pl.BlockSpec((1, n, 8), lambda i: (i, 0, 0)),
        out_shape=jax.ShapeDtypeStruct((b, n, 8), jnp.float32),
    )(*args)


def kernel(cls0, loc0, cls1, loc1, cls2, loc2, cls3, loc3, cls4, loc4):
    clss = (cls0, cls1, cls2, cls3, cls4)
    locs = (loc0, loc1, loc2, loc3, loc4)
    b = cls0.shape[0]

    cls_rows = []
    loc_rows = []
    max_parts = []
    for lvl, hw in enumerate(_HWS):
        p = hw * hw
        cf = clss[lvl].reshape(b, _A * _C, p)
        lf = locs[lvl].reshape(b, _A * 4, p)
        cls_rows.append(jnp.transpose(cf, (0, 2, 1)).reshape(b, p * _A, _C))
        loc_rows.append(jnp.transpose(lf, (0, 2, 1)).reshape(b, p * _A, 4))
        m = _anchor_max(cf, p)
        max_parts.append(jnp.transpose(m, (0, 2, 1)).reshape(b, p * _A))
    maxes = jnp.concatenate(max_parts, axis=1)

    aidx = _select_anchors(maxes)

    cls_cand = jnp.zeros((b, _NCAND, _C), jnp.float32)
    loc_cand = jnp.zeros((b, _NCAND, 4), jnp.float32)
    for lvl in range(len(_HWS)):
        klvl = _KS[lvl]
        g = aidx - _OFFS[lvl]
        in_lvl = (aidx >= _OFFS[lvl]) & (aidx < _OFFS[lvl + 1])
        a_loc = jnp.clip(g, 0, klvl - 1)
        gath_c = jnp.take_along_axis(cls_rows[lvl], a_loc[..., None], axis=1)
        gath_l = jnp.take_along_axis(loc_rows[lvl], a_loc[..., None], axis=1)
        cls_cand = cls_cand + jnp.where(in_lvl[..., None], gath_c, 0.0)
        loc_cand = loc_cand + jnp.where(in_lvl[..., None], gath_l, 0.0)

    anch_tab = jnp.asarray(_ANCHOR_TABLE)
    anch_cand = anch_tab[aidx]

    scores_cand = jax.nn.sigmoid(cls_cand).reshape(b, _NCAND * _C)
    ts, ti = jax.lax.top_k(scores_cand, 300)
    ci = ti // _C
    lbl = ti % _C

    sel_deltas = jnp.take_along_axis(loc_cand, ci[..., None], axis=1)
    sel_anch = jnp.take_along_axis(anch_cand, ci[..., None], axis=1)

    pad = _NMS_N - 300
    padf = lambda a, v: jnp.concatenate(
        [a, jnp.full((b, pad), v, jnp.float32)], axis=1)
    dx = padf(sel_deltas[..., 0], 0.0)
    dy = padf(sel_deltas[..., 1], 0.0)
    dw = padf(sel_deltas[..., 2], 0.0)
    dh = padf(sel_deltas[..., 3], 0.0)
    ax1 = padf(sel_anch[..., 0], 0.0)
    ay1 = padf(sel_anch[..., 1], 0.0)
    ax2 = padf(sel_anch[..., 2], 0.0)
    ay2 = padf(sel_anch[..., 3], 0.0)
    s = padf(ts, -1e9)
    lab = padf(lbl.astype(jnp.float32), 100.0)

    out = _nms_topk(dx, dy, dw, dh, ax1, ay1, ax2, ay2, s, lab)
    return out[:, :_POST_NMS, :6]


# speedup vs baseline: 1.0284x; 1.0284x over previous
import functools

import jax
import jax.numpy as jnp
import numpy as np
from jax.experimental import pallas as pl

_STRIDES = (8, 16, 32, 64, 128)
_HWS = (64, 32, 16, 8, 4)
_A = 9
_C = 80
_IMG = 512.0
_DWH_CLAMP = 4.135
_IOU_THR = 0.5
_POST_NMS = 100
_NCAND = 384
_NMS_N = 384

_KS = tuple(hw * hw * _A for hw in _HWS)
_OFFS = tuple(int(x) for x in np.concatenate([[0], np.cumsum(_KS)]))
_KTOT = _OFFS[-1]


def _make_anchor_table():
    tabs = []
    for hw, stride in zip(_HWS, _STRIDES):
        ratios = np.array([0.5, 1.0, 2.0])
        scales = np.array([2.0 ** 0.0, 2.0 ** (1.0 / 3.0), 2.0 ** (2.0 / 3.0)])
        base = 4.0 * stride
        ws, hs = [], []
        for r in ratios:
            for s in scales:
                size = base * s
                ws.append(size * np.sqrt(1.0 / r))
                hs.append(size * np.sqrt(r))
        ws = np.array(ws)
        hs = np.array(hs)
        xs = (np.arange(hw) + 0.5) * stride
        ys = (np.arange(hw) + 0.5) * stride
        cx, cy = np.meshgrid(xs, ys)
        cx = cx.reshape(-1, 1)
        cy = cy.reshape(-1, 1)
        x1 = cx - ws / 2.0
        y1 = cy - hs / 2.0
        x2 = cx + ws / 2.0
        y2 = cy + hs / 2.0
        tabs.append(np.stack([x1, y1, x2, y2], axis=-1).reshape(-1, 4))
    return np.concatenate(tabs, axis=0).astype(np.float32)


_ANCHOR_TABLE = _make_anchor_table()


def _amax_body(x_ref, o_ref):
    x = x_ref[0]
    outs = []
    for j in range(_A):
        outs.append(jnp.max(x[j * _C:(j + 1) * _C, :], axis=0))
    o_ref[0] = jnp.stack(outs, axis=0)


def _anchor_max(cls_flat, p):
    bp = min(p, 512)
    grid = (cls_flat.shape[0], p // bp)
    return pl.pallas_call(
        _amax_body,
        grid=grid,
        in_specs=[pl.BlockSpec((1, _A * _C, bp), lambda b, q: (b, 0, q))],
        out_specs=pl.BlockSpec((1, _A, bp), lambda b, q: (b, 0, q)),
        out_shape=jax.ShapeDtypeStruct((cls_flat.shape[0], _A, p), jnp.float32),
    )(cls_flat)


def _orderable_ukey(v):
    i32 = jax.lax.bitcast_convert_type(v, jnp.int32)
    key = i32 ^ jnp.bitwise_and(jnp.right_shift(i32, 31),
                                jnp.int32(0x7FFFFFFF))
    return jax.lax.bitcast_convert_type(key, jnp.uint32) ^ jnp.uint32(
        0x80000000)


def _compact_selected(gt, eq, quota, k):
    rr = gt.shape[0]
    up_tri = (jax.lax.broadcasted_iota(jnp.int32, (128, 128), 0)
              < jax.lax.broadcasted_iota(jnp.int32, (128, 128), 1)
              ).astype(jnp.float32)
    low_r = (jax.lax.broadcasted_iota(jnp.int32, (rr, rr), 0)
             < jax.lax.broadcasted_iota(jnp.int32, (rr, rr), 1)
             ).astype(jnp.float32)

    eq_f = eq.astype(jnp.float32)
    e_in = jax.lax.dot(eq_f, up_tri, preferred_element_type=jnp.float32)
    s_eq = jnp.sum(eq_f, axis=1, keepdims=True)
    s_pre = jnp.transpose(jnp.sum(low_r * s_eq, axis=0, keepdims=True))
    eqrank = s_pre + e_in
    sel = gt | (eq & (eqrank < quota))

    sel_f = sel.astype(jnp.float32)
    p_in = jax.lax.dot(sel_f, up_tri, preferred_element_type=jnp.float32)
    t_cnt = jnp.sum(sel_f, axis=1, keepdims=True)
    t_pre = jnp.transpose(jnp.sum(low_r * t_cnt, axis=0, keepdims=True))

    trow = jnp.transpose(t_pre)
    pcol = jax.lax.broadcasted_iota(
        jnp.int32, (k, 1), 0).astype(jnp.float32)
    row_of_p = jnp.sum((trow <= pcol).astype(jnp.float32),
                       axis=1, keepdims=True) - 1.0
    ridx = jax.lax.broadcasted_iota(
        jnp.int32, (k, rr), 1).astype(jnp.float32)
    oh = (ridx == row_of_p).astype(jnp.float32)
    t_g = jnp.sum(oh * trow, axis=1, keepdims=True)
    o_p = pcol - t_g
    w_g = jax.lax.dot(oh, p_in, preferred_element_type=jnp.float32)
    sel_g = jax.lax.dot(oh, sel_f, preferred_element_type=jnp.float32)
    lane = jax.lax.broadcasted_iota(
        jnp.int32, (k, 128), 1).astype(jnp.float32)
    hit = sel_g * (w_g == o_p).astype(jnp.float32)
    out = jnp.sum(hit * (row_of_p * 128.0 + lane), axis=1, keepdims=True)
    return out.astype(jnp.int32)


def _radix_threshold(v3, valid, k):
    bsz, rr, _ = v3.shape
    ukey = _orderable_ukey(v3)
    row_i = jax.lax.broadcasted_iota(jnp.int32, (bsz, rr, 128), 1)
    col_i = jax.lax.broadcasted_iota(jnp.int32, (bsz, rr, 128), 2)
    flat = row_i * 128 + col_i
    ukey = jnp.where(flat < valid, ukey, jnp.uint32(0))

    t = jnp.zeros((bsz, 1, 1), jnp.uint32)
    for bit in range(31, -1, -1):
        cand = t | jnp.uint32(1 << bit)
        cnt = jnp.sum((ukey >= cand).astype(jnp.float32), axis=(1, 2),
                      keepdims=True)
        t = jnp.where(cnt >= k, cand, t)

    gt3 = ukey > t
    eq3 = ukey == t
    quota = jnp.float32(k) - jnp.sum(gt3.astype(jnp.float32), axis=(1, 2),
                                     keepdims=True)
    return gt3, eq3, quota


def _select_body(v_ref, o_ref):
    v3 = v_ref[...]
    gt3, eq3, quota = _radix_threshold(v3, _KTOT, _NCAND)
    for bidx in range(v3.shape[0]):
        idx = _compact_selected(gt3[bidx], eq3[bidx], quota[bidx], _NCAND)
        o_ref[bidx] = jnp.transpose(idx)


def _select_anchors(maxes):
    b = maxes.shape[0]
    npad = 128 * ((_KTOT + 127) // 128) - _KTOT
    mp = jnp.concatenate(
        [maxes, jnp.zeros((b, npad), jnp.float32)], axis=1)
    rr = (_KTOT + npad) // 128
    mp = mp.reshape(b, rr, 128)
    out = pl.pallas_call(
        _select_body,
        grid=(1,),
        in_specs=[pl.BlockSpec((b, rr, 128), lambda i: (0, 0, 0))],
        out_specs=pl.BlockSpec((b, 1, _NCAND), lambda i: (0, 0, 0)),
        out_shape=jax.ShapeDtypeStruct((b, 1, _NCAND), jnp.int32),
    )(mp)
    return out.reshape(b, _NCAND)


def _nms_body(dx_ref, dy_ref, dw_ref, dh_ref, ax1_ref, ay1_ref, ax2_ref,
              ay2_ref, s_ref, lab_ref, o_ref):
    dx = dx_ref[0]
    dy = dy_ref[0]
    dw = jnp.clip(dw_ref[0], -_DWH_CLAMP, _DWH_CLAMP)
    dh = jnp.clip(dh_ref[0], -_DWH_CLAMP, _DWH_CLAMP)
    ax1 = ax1_ref[0]
    ay1 = ay1_ref[0]
    ax2 = ax2_ref[0]
    ay2 = ay2_ref[0]
    s = s_ref[0]
    lab = lab_ref[0]

    wa = ax2 - ax1
    ha = ay2 - ay1
    xa = ax1 + 0.5 * wa
    ya = ay1 + 0.5 * ha
    px = dx * wa + xa
    py = dy * ha + ya
    pw = jnp.exp(dw) * wa
    ph = jnp.exp(dh) * ha
    x1 = jnp.clip(px - 0.5 * pw, 0.0, _IMG)
    y1 = jnp.clip(py - 0.5 * ph, 0.0, _IMG)
    x2 = jnp.clip(px + 0.5 * pw, 0.0, _IMG)
    y2 = jnp.clip(py + 0.5 * ph, 0.0, _IMG)

    off = lab * (2.0 * _IMG)
    ox1 = x1 + off
    oy1 = y1 + off
    ox2 = x2 + off
    oy2 = y2 + off

    area = (x2 - x1) * (y2 - y1)

    cx1 = jnp.transpose(ox1)
    cy1 = jnp.transpose(oy1)
    cx2 = jnp.transpose(ox2)
    cy2 = jnp.transpose(oy2)
    carea = jnp.transpose(area)
    cs = jnp.transpose(s)

    ltx = jnp.maximum(cx1, ox1)
    lty = jnp.maximum(cy1, oy1)
    rbx = jnp.minimum(cx2, ox2)
    rby = jnp.minimum(cy2, oy2)
    w = jnp.clip(rbx - ltx, 0.0, None)
    h = jnp.clip(rby - lty, 0.0, None)
    inter = w * h
    union = carea + area - inter
    iou = inter / jnp.clip(union, 1e-6, None)

    higher = cs > s
    sup = jnp.any(higher & (iou > _IOU_THR), axis=0, keepdims=True)

    s2 = jnp.where(sup, -1e9, s)

    in_a = jnp.logical_and(jnp.logical_not(sup), s > 0.0)
    n = s.shape[1]
    ia_row = in_a
    ia_col = jnp.transpose(in_a)
    row_j = jax.lax.broadcasted_iota(jnp.int32, (n, n), 1)
    col_i = jax.lax.broadcasted_iota(jnp.int32, (n, n), 0)
    before = jnp.logical_or(
        jnp.logical_and(ia_col, jnp.logical_not(ia_row)),
        jnp.logical_and(ia_col == ia_row, col_i < row_j),
    )
    pos = jnp.sum(before.astype(jnp.float32), axis=0, keepdims=True)

    sel = (pos == col_i.astype(jnp.float32)).astype(jnp.float32)

    def compact(row):
        return jnp.sum(sel * row, axis=1, keepdims=True)

    out = jnp.concatenate(
        [compact(x1), compact(y1), compact(x2), compact(y2),
         compact(s2), compact(lab), jnp.zeros((n, 2), jnp.float32)], axis=1)
    o_ref[0] = out


def _nms_topk(dx, dy, dw, dh, ax1, ay1, ax2, ay2, s, lab):
    b, n = s.shape
    r3 = lambda a: a.reshape(b, 1, n)
    args = [r3(a) for a in (dx, dy, dw, dh, ax1, ay1, ax2, ay2, s, lab)]
    spec = pl.BlockSpec((1, 1, n), lambda i: (i, 0, 0))
    return pl.pallas_call(
        _nms_body,
        grid=(b,),
        in_specs=[spec] * 10,
        out_specs=pl.BlockSpec((1, n, 8), lambda i: (i, 0, 0)),
        out_shape=jax.ShapeDtypeStruct((b, n, 8), jnp.float32),
    )(*args)


def kernel(cls0, loc0, cls1, loc1, cls2, loc2, cls3, loc3, cls4, loc4):
    clss = (cls0, cls1, cls2, cls3, cls4)
    locs = (loc0, loc1, loc2, loc3, loc4)
    b = cls0.shape[0]

    cls_rows = []
    loc_rows = []
    max_parts = []
    for lvl, hw in enumerate(_HWS):
        p = hw * hw
        cf = clss[lvl].reshape(b, _A * _C, p)
        lf = locs[lvl].reshape(b, _A * 4, p)
        cls_rows.append(jnp.transpose(cf, (0, 2, 1)).reshape(b, p * _A, _C))
        loc_rows.append(jnp.transpose(lf, (0, 2, 1)).reshape(b, p * _A, 4))
        m = _anchor_max(cf, p)
        max_parts.append(jnp.transpose(m, (0, 2, 1)).reshape(b, p * _A))
    maxes = jnp.concatenate(max_parts, axis=1)

    aidx = _select_anchors(maxes)

    cls_cand = jnp.zeros((b, _NCAND, _C), jnp.float32)
    loc_cand = jnp.zeros((b, _NCAND, 4), jnp.float32)
    for lvl in range(len(_HWS)):
        klvl = _KS[lvl]
        g = aidx - _OFFS[lvl]
        in_lvl = (aidx >= _OFFS[lvl]) & (aidx < _OFFS[lvl + 1])
        a_loc = jnp.clip(g, 0, klvl - 1)
        gath_c = jnp.take_along_axis(cls_rows[lvl], a_loc[..., None], axis=1)
        gath_l = jnp.take_along_axis(loc_rows[lvl], a_loc[..., None], axis=1)
        cls_cand = cls_cand + jnp.where(in_lvl[..., None], gath_c, 0.0)
        loc_cand = loc_cand + jnp.where(in_lvl[..., None], gath_l, 0.0)

    anch_tab = jnp.asarray(_ANCHOR_TABLE)
    anch_cand = anch_tab[aidx]

    scores_cand = jax.nn.sigmoid(cls_cand).reshape(b, _NCAND * _C)
    ts, ti = jax.lax.top_k(scores_cand, 300)
    ci = ti // _C
    lbl = ti % _C

    sel_deltas = jnp.take_along_axis(loc_cand, ci[..., None], axis=1)
    sel_anch = jnp.take_along_axis(anch_cand, ci[..., None], axis=1)

    pad = _NMS_N - 300
    padf = lambda a, v: jnp.concatenate(
        [a, jnp.full((b, pad), v, jnp.float32)], axis=1)
    dx = padf(sel_deltas[..., 0], 0.0)
    dy = padf(sel_deltas[..., 1], 0.0)
    dw = padf(sel_deltas[..., 2], 0.0)
    dh = padf(sel_deltas[..., 3], 0.0)
    ax1 = padf(sel_anch[..., 0], 0.0)
    ay1 = padf(sel_anch[..., 1], 0.0)
    ax2 = padf(sel_anch[..., 2], 0.0)
    ay2 = padf(sel_anch[..., 3], 0.0)
    s = padf(ts, -1e9)
    lab = padf(lbl.astype(jnp.float32), 100.0)

    out = _nms_topk(dx, dy, dw, dh, ax1, ay1, ax2, ay2, s, lab)
    return out[:, :_POST_NMS, :6]


# speedup vs baseline: 1.1045x; 1.0740x over previous
import functools

import jax
import jax.numpy as jnp
import numpy as np
from jax.experimental import pallas as pl

_STRIDES = (8, 16, 32, 64, 128)
_HWS = (64, 32, 16, 8, 4)
_A = 9
_C = 80
_IMG = 512.0
_DWH_CLAMP = 4.135
_IOU_THR = 0.5
_POST_NMS = 100
_NCAND = 384
_NMS_N = 384

_KS = tuple(hw * hw * _A for hw in _HWS)
_OFFS = tuple(int(x) for x in np.concatenate([[0], np.cumsum(_KS)]))
_KTOT = _OFFS[-1]


def _make_anchor_table():
    tabs = []
    for hw, stride in zip(_HWS, _STRIDES):
        ratios = np.array([0.5, 1.0, 2.0])
        scales = np.array([2.0 ** 0.0, 2.0 ** (1.0 / 3.0), 2.0 ** (2.0 / 3.0)])
        base = 4.0 * stride
        ws, hs = [], []
        for r in ratios:
            for s in scales:
                size = base * s
                ws.append(size * np.sqrt(1.0 / r))
                hs.append(size * np.sqrt(r))
        ws = np.array(ws)
        hs = np.array(hs)
        xs = (np.arange(hw) + 0.5) * stride
        ys = (np.arange(hw) + 0.5) * stride
        cx, cy = np.meshgrid(xs, ys)
        cx = cx.reshape(-1, 1)
        cy = cy.reshape(-1, 1)
        x1 = cx - ws / 2.0
        y1 = cy - hs / 2.0
        x2 = cx + ws / 2.0
        y2 = cy + hs / 2.0
        tabs.append(np.stack([x1, y1, x2, y2], axis=-1).reshape(-1, 4))
    return np.concatenate(tabs, axis=0).astype(np.float32)


_ANCHOR_TABLE = _make_anchor_table()


def _amax_body(x_ref, o_ref):
    x = x_ref[0]
    outs = []
    for j in range(_A):
        outs.append(jnp.max(x[j * _C:(j + 1) * _C, :], axis=0))
    o_ref[0] = jnp.stack(outs, axis=0)


def _anchor_max(cls_flat, p):
    bp = min(p, 512)
    grid = (cls_flat.shape[0], p // bp)
    return pl.pallas_call(
        _amax_body,
        grid=grid,
        in_specs=[pl.BlockSpec((1, _A * _C, bp), lambda b, q: (b, 0, q))],
        out_specs=pl.BlockSpec((1, _A, bp), lambda b, q: (b, 0, q)),
        out_shape=jax.ShapeDtypeStruct((cls_flat.shape[0], _A, p), jnp.float32),
    )(cls_flat)


def _orderable_ukey(v):
    i32 = jax.lax.bitcast_convert_type(v, jnp.int32)
    key = i32 ^ jnp.bitwise_and(jnp.right_shift(i32, 31),
                                jnp.int32(0x7FFFFFFF))
    return jax.lax.bitcast_convert_type(key, jnp.uint32) ^ jnp.uint32(
        0x80000000)


def _compact_selected(gt, eq, quota, k):
    rr = gt.shape[0]
    up_tri = (jax.lax.broadcasted_iota(jnp.int32, (128, 128), 0)
              < jax.lax.broadcasted_iota(jnp.int32, (128, 128), 1)
              ).astype(jnp.float32)
    low_r = (jax.lax.broadcasted_iota(jnp.int32, (rr, rr), 0)
             < jax.lax.broadcasted_iota(jnp.int32, (rr, rr), 1)
             ).astype(jnp.float32)

    eq_f = eq.astype(jnp.float32)
    e_in = jax.lax.dot(eq_f, up_tri, preferred_element_type=jnp.float32)
    s_eq = jnp.sum(eq_f, axis=1, keepdims=True)
    s_pre = jnp.transpose(jnp.sum(low_r * s_eq, axis=0, keepdims=True))
    eqrank = s_pre + e_in
    sel = gt | (eq & (eqrank < quota))

    sel_f = sel.astype(jnp.float32)
    p_in = jax.lax.dot(sel_f, up_tri, preferred_element_type=jnp.float32)
    t_cnt = jnp.sum(sel_f, axis=1, keepdims=True)
    t_pre = jnp.transpose(jnp.sum(low_r * t_cnt, axis=0, keepdims=True))

    trow = jnp.transpose(t_pre)
    pcol = jax.lax.broadcasted_iota(
        jnp.int32, (k, 1), 0).astype(jnp.float32)
    row_of_p = jnp.sum((trow <= pcol).astype(jnp.float32),
                       axis=1, keepdims=True) - 1.0
    ridx = jax.lax.broadcasted_iota(
        jnp.int32, (k, rr), 1).astype(jnp.float32)
    oh = (ridx == row_of_p).astype(jnp.float32)
    t_g = jnp.sum(oh * trow, axis=1, keepdims=True)
    o_p = pcol - t_g
    w_g = jax.lax.dot(oh, p_in, preferred_element_type=jnp.float32)
    sel_g = jax.lax.dot(oh, sel_f, preferred_element_type=jnp.float32)
    lane = jax.lax.broadcasted_iota(
        jnp.int32, (k, 128), 1).astype(jnp.float32)
    hit = sel_g * (w_g == o_p).astype(jnp.float32)
    out = jnp.sum(hit * (row_of_p * 128.0 + lane), axis=1, keepdims=True)
    return out.astype(jnp.int32)


def _radix_threshold(v3, valid, k):
    bsz, rr, _ = v3.shape
    ukey = _orderable_ukey(v3)
    row_i = jax.lax.broadcasted_iota(jnp.int32, (bsz, rr, 128), 1)
    col_i = jax.lax.broadcasted_iota(jnp.int32, (bsz, rr, 128), 2)
    flat = row_i * 128 + col_i
    ukey = jnp.where(flat < valid, ukey, jnp.uint32(0))

    t = jnp.zeros((bsz, 1, 1), jnp.uint32)
    for bit in range(31, -1, -1):
        cand = t | jnp.uint32(1 << bit)
        cnt = jnp.sum((ukey >= cand).astype(jnp.float32), axis=(1, 2),
                      keepdims=True)
        t = jnp.where(cnt >= k, cand, t)

    gt3 = ukey > t
    eq3 = ukey == t
    quota = jnp.float32(k) - jnp.sum(gt3.astype(jnp.float32), axis=(1, 2),
                                     keepdims=True)
    return gt3, eq3, quota


def _select_body(v_ref, o_ref):
    v3 = v_ref[...]
    gt3, eq3, quota = _radix_threshold(v3, _KTOT, _NCAND)
    for bidx in range(v3.shape[0]):
        idx = _compact_selected(gt3[bidx], eq3[bidx], quota[bidx], _NCAND)
        o_ref[bidx] = jnp.transpose(idx)


def _select_anchors(maxes):
    b = maxes.shape[0]
    npad = 128 * ((_KTOT + 127) // 128) - _KTOT
    mp = jnp.concatenate(
        [maxes, jnp.zeros((b, npad), jnp.float32)], axis=1)
    rr = (_KTOT + npad) // 128
    mp = mp.reshape(b, rr, 128)
    out = pl.pallas_call(
        _select_body,
        grid=(1,),
        in_specs=[pl.BlockSpec((b, rr, 128), lambda i: (0, 0, 0))],
        out_specs=pl.BlockSpec((b, 1, _NCAND), lambda i: (0, 0, 0)),
        out_shape=jax.ShapeDtypeStruct((b, 1, _NCAND), jnp.int32),
    )(mp)
    return out.reshape(b, _NCAND)


def _nms_body(dx_ref, dy_ref, dw_ref, dh_ref, ax1_ref, ay1_ref, ax2_ref,
              ay2_ref, s_ref, lab_ref, o_ref):
    dx = dx_ref[0]
    dy = dy_ref[0]
    dw = jnp.clip(dw_ref[0], -_DWH_CLAMP, _DWH_CLAMP)
    dh = jnp.clip(dh_ref[0], -_DWH_CLAMP, _DWH_CLAMP)
    ax1 = ax1_ref[0]
    ay1 = ay1_ref[0]
    ax2 = ax2_ref[0]
    ay2 = ay2_ref[0]
    s = s_ref[0]
    lab = lab_ref[0]

    wa = ax2 - ax1
    ha = ay2 - ay1
    xa = ax1 + 0.5 * wa
    ya = ay1 + 0.5 * ha
    px = dx * wa + xa
    py = dy * ha + ya
    pw = jnp.exp(dw) * wa
    ph = jnp.exp(dh) * ha
    x1 = jnp.clip(px - 0.5 * pw, 0.0, _IMG)
    y1 = jnp.clip(py - 0.5 * ph, 0.0, _IMG)
    x2 = jnp.clip(px + 0.5 * pw, 0.0, _IMG)
    y2 = jnp.clip(py + 0.5 * ph, 0.0, _IMG)

    off = lab * (2.0 * _IMG)
    ox1 = x1 + off
    oy1 = y1 + off
    ox2 = x2 + off
    oy2 = y2 + off

    area = (x2 - x1) * (y2 - y1)

    cx1 = jnp.transpose(ox1)
    cy1 = jnp.transpose(oy1)
    cx2 = jnp.transpose(ox2)
    cy2 = jnp.transpose(oy2)
    carea = jnp.transpose(area)
    cs = jnp.transpose(s)

    ltx = jnp.maximum(cx1, ox1)
    lty = jnp.maximum(cy1, oy1)
    rbx = jnp.minimum(cx2, ox2)
    rby = jnp.minimum(cy2, oy2)
    w = jnp.clip(rbx - ltx, 0.0, None)
    h = jnp.clip(rby - lty, 0.0, None)
    inter = w * h
    union = carea + area - inter
    iou = inter / jnp.clip(union, 1e-6, None)

    higher = cs > s
    sup = jnp.any(higher & (iou > _IOU_THR), axis=0, keepdims=True)

    s2 = jnp.where(sup, -1e9, s)

    in_a = jnp.logical_and(jnp.logical_not(sup), s > 0.0)
    n = s.shape[1]
    ia_row = in_a
    ia_col = jnp.transpose(in_a)
    row_j = jax.lax.broadcasted_iota(jnp.int32, (n, n), 1)
    col_i = jax.lax.broadcasted_iota(jnp.int32, (n, n), 0)
    before = jnp.logical_or(
        jnp.logical_and(ia_col, jnp.logical_not(ia_row)),
        jnp.logical_and(ia_col == ia_row, col_i < row_j),
    )
    pos = jnp.sum(before.astype(jnp.float32), axis=0, keepdims=True)

    sel = (pos == col_i.astype(jnp.float32)).astype(jnp.float32)

    def compact(row):
        return jnp.sum(sel * row, axis=1, keepdims=True)

    out = jnp.concatenate(
        [compact(x1), compact(y1), compact(x2), compact(y2),
         compact(s2), compact(lab), jnp.zeros((n, 2), jnp.float32)], axis=1)
    o_ref[0] = out


def _nms_topk(dx, dy, dw, dh, ax1, ay1, ax2, ay2, s, lab):
    b, n = s.shape
    r3 = lambda a: a.reshape(b, 1, n)
    args = [r3(a) for a in (dx, dy, dw, dh, ax1, ay1, ax2, ay2, s, lab)]
    spec = pl.BlockSpec((1, 1, n), lambda i: (i, 0, 0))
    return pl.pallas_call(
        _nms_body,
        grid=(b,),
        in_specs=[spec] * 10,
        out_specs=pl.BlockSpec((1, n, 8), lambda i: (i, 0, 0)),
        out_shape=jax.ShapeDtypeStruct((b, n, 8), jnp.float32),
    )(*args)


def kernel(cls0, loc0, cls1, loc1, cls2, loc2, cls3, loc3, cls4, loc4):
    clss = (cls0, cls1, cls2, cls3, cls4)
    locs = (loc0, loc1, loc2, loc3, loc4)
    b = cls0.shape[0]

    cls_rows = []
    loc_rows = []
    max_parts = []
    for lvl, hw in enumerate(_HWS):
        p = hw * hw
        cf = clss[lvl].reshape(b, _A * _C, p)
        lf = locs[lvl].reshape(b, _A * 4, p)
        cls_rows.append(jnp.transpose(cf, (0, 2, 1)).reshape(b, p * _A, _C))
        loc_rows.append(jnp.transpose(lf, (0, 2, 1)).reshape(b, p * _A, 4))
        m = _anchor_max(cf, p)
        max_parts.append(jnp.transpose(m, (0, 2, 1)).reshape(b, p * _A))
    maxes = jnp.concatenate(max_parts, axis=1)

    aidx = _select_anchors(maxes)

    crows = jnp.concatenate(cls_rows, axis=1)
    lrows = jnp.concatenate(loc_rows, axis=1)
    cls_cand = jnp.take_along_axis(crows, aidx[..., None], axis=1)
    loc_cand = jnp.take_along_axis(lrows, aidx[..., None], axis=1)

    anch_tab = jnp.asarray(_ANCHOR_TABLE)
    anch_cand = anch_tab[aidx]

    scores_cand = jax.nn.sigmoid(cls_cand).reshape(b, _NCAND * _C)
    ts, ti = jax.lax.top_k(scores_cand, 300)
    ci = ti // _C
    lbl = ti % _C

    sel_deltas = jnp.take_along_axis(loc_cand, ci[..., None], axis=1)
    sel_anch = jnp.take_along_axis(anch_cand, ci[..., None], axis=1)

    pad = _NMS_N - 300
    padf = lambda a, v: jnp.concatenate(
        [a, jnp.full((b, pad), v, jnp.float32)], axis=1)
    dx = padf(sel_deltas[..., 0], 0.0)
    dy = padf(sel_deltas[..., 1], 0.0)
    dw = padf(sel_deltas[..., 2], 0.0)
    dh = padf(sel_deltas[..., 3], 0.0)
    ax1 = padf(sel_anch[..., 0], 0.0)
    ay1 = padf(sel_anch[..., 1], 0.0)
    ax2 = padf(sel_anch[..., 2], 0.0)
    ay2 = padf(sel_anch[..., 3], 0.0)
    s = padf(ts, -1e9)
    lab = padf(lbl.astype(jnp.float32), 100.0)

    out = _nms_topk(dx, dy, dw, dh, ax1, ay1, ax2, ay2, s, lab)
    return out[:, :_POST_NMS, :6]


# speedup vs baseline: 1.3266x; 1.2011x over previous
import functools

import jax
import jax.numpy as jnp
import numpy as np
from jax.experimental import pallas as pl

_STRIDES = (8, 16, 32, 64, 128)
_HWS = (64, 32, 16, 8, 4)
_A = 9
_C = 80
_IMG = 512.0
_DWH_CLAMP = 4.135
_IOU_THR = 0.5
_POST_NMS = 100
_NCAND = 384
_NMS_N = 384

_KS = tuple(hw * hw * _A for hw in _HWS)
_OFFS = tuple(int(x) for x in np.concatenate([[0], np.cumsum(_KS)]))
_KTOT = _OFFS[-1]


def _make_anchor_table():
    tabs = []
    for hw, stride in zip(_HWS, _STRIDES):
        ratios = np.array([0.5, 1.0, 2.0])
        scales = np.array([2.0 ** 0.0, 2.0 ** (1.0 / 3.0), 2.0 ** (2.0 / 3.0)])
        base = 4.0 * stride
        ws, hs = [], []
        for r in ratios:
            for s in scales:
                size = base * s
                ws.append(size * np.sqrt(1.0 / r))
                hs.append(size * np.sqrt(r))
        ws = np.array(ws)
        hs = np.array(hs)
        xs = (np.arange(hw) + 0.5) * stride
        ys = (np.arange(hw) + 0.5) * stride
        cx, cy = np.meshgrid(xs, ys)
        cx = cx.reshape(-1, 1)
        cy = cy.reshape(-1, 1)
        x1 = cx - ws / 2.0
        y1 = cy - hs / 2.0
        x2 = cx + ws / 2.0
        y2 = cy + hs / 2.0
        tabs.append(np.stack([x1, y1, x2, y2], axis=-1).reshape(-1, 4))
    return np.concatenate(tabs, axis=0).astype(np.float32)


_ANCHOR_TABLE = _make_anchor_table()


def _amax_body(x_ref, o_ref):
    x = x_ref[0]
    outs = []
    for j in range(_A):
        outs.append(jnp.max(x[j * _C:(j + 1) * _C, :], axis=0))
    o_ref[0] = jnp.stack(outs, axis=0)


def _anchor_max(cls_flat, p):
    bp = min(p, 512)
    grid = (cls_flat.shape[0], p // bp)
    return pl.pallas_call(
        _amax_body,
        grid=grid,
        in_specs=[pl.BlockSpec((1, _A * _C, bp), lambda b, q: (b, 0, q))],
        out_specs=pl.BlockSpec((1, _A, bp), lambda b, q: (b, 0, q)),
        out_shape=jax.ShapeDtypeStruct((cls_flat.shape[0], _A, p), jnp.float32),
    )(cls_flat)


def _orderable_ukey(v):
    i32 = jax.lax.bitcast_convert_type(v, jnp.int32)
    key = i32 ^ jnp.bitwise_and(jnp.right_shift(i32, 31),
                                jnp.int32(0x7FFFFFFF))
    return jax.lax.bitcast_convert_type(key, jnp.uint32) ^ jnp.uint32(
        0x80000000)


def _compact_selected(gt, eq, quota, k):
    rr = gt.shape[0]
    up_tri = (jax.lax.broadcasted_iota(jnp.int32, (128, 128), 0)
              < jax.lax.broadcasted_iota(jnp.int32, (128, 128), 1)
              ).astype(jnp.float32)
    low_r = (jax.lax.broadcasted_iota(jnp.int32, (rr, rr), 0)
             < jax.lax.broadcasted_iota(jnp.int32, (rr, rr), 1)
             ).astype(jnp.float32)

    eq_f = eq.astype(jnp.float32)
    e_in = jax.lax.dot(eq_f, up_tri, preferred_element_type=jnp.float32)
    s_eq = jnp.sum(eq_f, axis=1, keepdims=True)
    s_pre = jnp.transpose(jnp.sum(low_r * s_eq, axis=0, keepdims=True))
    eqrank = s_pre + e_in
    sel = gt | (eq & (eqrank < quota))

    sel_f = sel.astype(jnp.float32)
    p_in = jax.lax.dot(sel_f, up_tri, preferred_element_type=jnp.float32)
    t_cnt = jnp.sum(sel_f, axis=1, keepdims=True)
    t_pre = jnp.transpose(jnp.sum(low_r * t_cnt, axis=0, keepdims=True))

    trow = jnp.transpose(t_pre)
    pcol = jax.lax.broadcasted_iota(
        jnp.int32, (k, 1), 0).astype(jnp.float32)
    row_of_p = jnp.sum((trow <= pcol).astype(jnp.float32),
                       axis=1, keepdims=True) - 1.0
    ridx = jax.lax.broadcasted_iota(
        jnp.int32, (k, rr), 1).astype(jnp.float32)
    oh = (ridx == row_of_p).astype(jnp.float32)
    t_g = jnp.sum(oh * trow, axis=1, keepdims=True)
    o_p = pcol - t_g
    w_g = jax.lax.dot(oh, p_in, preferred_element_type=jnp.float32)
    sel_g = jax.lax.dot(oh, sel_f, preferred_element_type=jnp.float32)
    lane = jax.lax.broadcasted_iota(
        jnp.int32, (k, 128), 1).astype(jnp.float32)
    hit = sel_g * (w_g == o_p).astype(jnp.float32)
    out = jnp.sum(hit * (row_of_p * 128.0 + lane), axis=1, keepdims=True)
    return out.astype(jnp.int32)


def _radix_threshold(v3, valid, k):
    bsz, rr, _ = v3.shape
    ukey = _orderable_ukey(v3)
    row_i = jax.lax.broadcasted_iota(jnp.int32, (bsz, rr, 128), 1)
    col_i = jax.lax.broadcasted_iota(jnp.int32, (bsz, rr, 128), 2)
    flat = row_i * 128 + col_i
    ukey = jnp.where(flat < valid, ukey, jnp.uint32(0))

    t = jnp.zeros((bsz, 1, 1), jnp.uint32)
    for bit in range(31, -1, -1):
        cand = t | jnp.uint32(1 << bit)
        cnt = jnp.sum((ukey >= cand).astype(jnp.float32), axis=(1, 2),
                      keepdims=True)
        t = jnp.where(cnt >= k, cand, t)

    gt3 = ukey > t
    eq3 = ukey == t
    quota = jnp.float32(k) - jnp.sum(gt3.astype(jnp.float32), axis=(1, 2),
                                     keepdims=True)
    return gt3, eq3, quota


def _select_body(v_ref, o_ref):
    v3 = v_ref[...]
    gt3, eq3, quota = _radix_threshold(v3, _KTOT, _NCAND)
    for bidx in range(v3.shape[0]):
        idx = _compact_selected(gt3[bidx], eq3[bidx], quota[bidx], _NCAND)
        o_ref[bidx] = jnp.transpose(idx)


def _select_anchors(maxes):
    b = maxes.shape[0]
    npad = 128 * ((_KTOT + 127) // 128) - _KTOT
    mp = jnp.concatenate(
        [maxes, jnp.zeros((b, npad), jnp.float32)], axis=1)
    rr = (_KTOT + npad) // 128
    mp = mp.reshape(b, rr, 128)
    out = pl.pallas_call(
        _select_body,
        grid=(1,),
        in_specs=[pl.BlockSpec((b, rr, 128), lambda i: (0, 0, 0))],
        out_specs=pl.BlockSpec((b, 1, _NCAND), lambda i: (0, 0, 0)),
        out_shape=jax.ShapeDtypeStruct((b, 1, _NCAND), jnp.int32),
    )(mp)
    return out.reshape(b, _NCAND)


def _nms_body(s_ref, dx_ref, dy_ref, dw_ref, dh_ref, ax1_ref, ay1_ref,
              ax2_ref, ay2_ref, o_ref):
    sc = s_ref[0]
    rr = sc.shape[0]
    gt3, eq3, quota3 = _radix_threshold(sc[None], rr * 128, 300)
    idx = _compact_selected(gt3[0], eq3[0], quota3[0], 300)
    idxf = idx.astype(jnp.float32)

    row_of = jnp.floor((idxf + 0.5) / 128.0)
    col_of = idxf - 128.0 * row_of
    ohr = (jax.lax.broadcasted_iota(jnp.int32, (300, rr), 1
                                    ).astype(jnp.float32) == row_of
           ).astype(jnp.float32)
    rows = jax.lax.dot(ohr, sc, preferred_element_type=jnp.float32)
    lane300 = jax.lax.broadcasted_iota(jnp.int32, (300, 128), 1
                                       ).astype(jnp.float32)
    v_sel = jnp.sum(rows * (lane300 == col_of).astype(jnp.float32),
                    axis=1, keepdims=True)
    cand = jnp.floor((idxf + 0.5) / float(_C))
    lbl = idxf - float(_C) * cand

    vrow = jnp.transpose(v_sel)
    qlt = (jax.lax.broadcasted_iota(jnp.int32, (300, 300), 0)
           < jax.lax.broadcasted_iota(jnp.int32, (300, 300), 1))
    beats = (v_sel > vrow) | ((v_sel == vrow) & qlt)
    rank = jnp.sum(beats.astype(jnp.float32), axis=0, keepdims=True)

    rcol = jax.lax.broadcasted_iota(jnp.int32, (_NMS_N, 300), 0
                                    ).astype(jnp.float32)
    ohs = (rcol == rank).astype(jnp.float32)

    def srt(col):
        return jax.lax.dot(ohs, col, preferred_element_type=jnp.float32)

    padmask = (jax.lax.broadcasted_iota(jnp.int32, (_NMS_N, 1), 0) >= 300)
    s_col = jnp.where(padmask, -1e9, srt(v_sel))
    lab_col = jnp.where(padmask, 100.0, srt(lbl))
    cand_col = srt(cand)

    oh2 = (jax.lax.broadcasted_iota(jnp.int32, (_NMS_N, _NCAND), 1
                                    ).astype(jnp.float32) == cand_col
           ).astype(jnp.float32)

    def g2(ref):
        return jnp.transpose(
            jnp.sum(oh2 * ref[0], axis=1, keepdims=True))

    dx = g2(dx_ref)
    dy = g2(dy_ref)
    dw = jnp.clip(g2(dw_ref), -_DWH_CLAMP, _DWH_CLAMP)
    dh = jnp.clip(g2(dh_ref), -_DWH_CLAMP, _DWH_CLAMP)
    ax1 = g2(ax1_ref)
    ay1 = g2(ay1_ref)
    ax2 = g2(ax2_ref)
    ay2 = g2(ay2_ref)
    s = jnp.transpose(s_col)
    lab = jnp.transpose(lab_col)

    wa = ax2 - ax1
    ha = ay2 - ay1
    xa = ax1 + 0.5 * wa
    ya = ay1 + 0.5 * ha
    px = dx * wa + xa
    py = dy * ha + ya
    pw = jnp.exp(dw) * wa
    ph = jnp.exp(dh) * ha
    x1 = jnp.clip(px - 0.5 * pw, 0.0, _IMG)
    y1 = jnp.clip(py - 0.5 * ph, 0.0, _IMG)
    x2 = jnp.clip(px + 0.5 * pw, 0.0, _IMG)
    y2 = jnp.clip(py + 0.5 * ph, 0.0, _IMG)

    off = lab * (2.0 * _IMG)
    ox1 = x1 + off
    oy1 = y1 + off
    ox2 = x2 + off
    oy2 = y2 + off

    area = (x2 - x1) * (y2 - y1)

    cx1 = jnp.transpose(ox1)
    cy1 = jnp.transpose(oy1)
    cx2 = jnp.transpose(ox2)
    cy2 = jnp.transpose(oy2)
    carea = jnp.transpose(area)
    cs = jnp.transpose(s)

    ltx = jnp.maximum(cx1, ox1)
    lty = jnp.maximum(cy1, oy1)
    rbx = jnp.minimum(cx2, ox2)
    rby = jnp.minimum(cy2, oy2)
    w = jnp.clip(rbx - ltx, 0.0, None)
    h = jnp.clip(rby - lty, 0.0, None)
    inter = w * h
    union = carea + area - inter
    iou = inter / jnp.clip(union, 1e-6, None)

    higher = cs > s
    sup = jnp.any(higher & (iou > _IOU_THR), axis=0, keepdims=True)

    s2 = jnp.where(sup, -1e9, s)

    in_a = jnp.logical_and(jnp.logical_not(sup), s > 0.0)
    n = s.shape[1]
    ia_row = in_a
    ia_col = jnp.transpose(in_a)
    row_j = jax.lax.broadcasted_iota(jnp.int32, (n, n), 1)
    col_i = jax.lax.broadcasted_iota(jnp.int32, (n, n), 0)
    before = jnp.logical_or(
        jnp.logical_and(ia_col, jnp.logical_not(ia_row)),
        jnp.logical_and(ia_col == ia_row, col_i < row_j),
    )
    pos = jnp.sum(before.astype(jnp.float32), axis=0, keepdims=True)

    sel = (pos == col_i.astype(jnp.float32)).astype(jnp.float32)

    def compact(row):
        return jnp.sum(sel * row, axis=1, keepdims=True)

    out = jnp.concatenate(
        [compact(x1), compact(y1), compact(x2), compact(y2),
         compact(s2), compact(lab), jnp.zeros((n, 2), jnp.float32)], axis=1)
    o_ref[0] = out


def _nms_topk(scores3, dx, dy, dw, dh, ax1, ay1, ax2, ay2):
    b, n = dx.shape
    r3 = lambda a: a.reshape(b, 1, n)
    args = [scores3] + [r3(a) for a in (dx, dy, dw, dh, ax1, ay1, ax2, ay2)]
    spec = pl.BlockSpec((1, 1, n), lambda i: (i, 0, 0))
    sspec = pl.BlockSpec((1, scores3.shape[1], 128), lambda i: (i, 0, 0))
    return pl.pallas_call(
        _nms_body,
        grid=(b,),
        in_specs=[sspec] + [spec] * 8,
        out_specs=pl.BlockSpec((1, n, 8), lambda i: (i, 0, 0)),
        out_shape=jax.ShapeDtypeStruct((b, n, 8), jnp.float32),
    )(*args)


def kernel(cls0, loc0, cls1, loc1, cls2, loc2, cls3, loc3, cls4, loc4):
    clss = (cls0, cls1, cls2, cls3, cls4)
    locs = (loc0, loc1, loc2, loc3, loc4)
    b = cls0.shape[0]

    cls_rows = []
    loc_rows = []
    max_parts = []
    for lvl, hw in enumerate(_HWS):
        p = hw * hw
        cf = clss[lvl].reshape(b, _A * _C, p)
        lf = locs[lvl].reshape(b, _A * 4, p)
        cls_rows.append(jnp.transpose(cf, (0, 2, 1)).reshape(b, p * _A, _C))
        loc_rows.append(jnp.transpose(lf, (0, 2, 1)).reshape(b, p * _A, 4))
        m = _anchor_max(cf, p)
        max_parts.append(jnp.transpose(m, (0, 2, 1)).reshape(b, p * _A))
    maxes = jnp.concatenate(max_parts, axis=1)

    aidx = _select_anchors(maxes)

    crows = jnp.concatenate(cls_rows, axis=1)
    lrows = jnp.concatenate(loc_rows, axis=1)
    cls_cand = jnp.take_along_axis(crows, aidx[..., None], axis=1)
    loc_cand = jnp.take_along_axis(lrows, aidx[..., None], axis=1)

    anch_tab = jnp.asarray(_ANCHOR_TABLE)
    anch_cand = anch_tab[aidx]

    scores3 = jax.nn.sigmoid(cls_cand).reshape(b, _NCAND * _C // 128, 128)
    out = _nms_topk(scores3,
                    loc_cand[..., 0], loc_cand[..., 1],
                    loc_cand[..., 2], loc_cand[..., 3],
                    anch_cand[..., 0], anch_cand[..., 1],
                    anch_cand[..., 2], anch_cand[..., 3])
    return out[:, :_POST_NMS, :6]
